# Initial kernel scaffold; baseline (speedup 1.0000x reference)
import jax
import jax.numpy as jnp
from jax.experimental import pallas as pl


def kernel(x, edge_index, Ws0, Wn0, b0, Ws1, Wn1, b1, Ws2, Wn2, b2):
    raise NotImplementedError("write your pallas kernel here")


# speedup vs baseline: 2.9303x; 2.9303x over previous
import functools

import jax
import jax.numpy as jnp
from jax import lax
from jax.experimental import pallas as pl
from jax.experimental.pallas import tpu as pltpu
from jax.experimental.pallas import tpu_sc as plsc

_N = 10000
_NP = 10240
_E = 160000
_D = 256
_H = 128
_NT = 16
_EPT = _E // _NT
_C = 80
_NCHUNK = _EPT // _C
_RPT = _NP // _NT
_BM = 1000


def _make_segsum(with_deg):
  mesh = plsc.VectorSubcoreMesh(core_axis_name="c", subcore_axis_name="s")
  if with_deg:
    out_type = [jax.ShapeDtypeStruct((_N, _H), jnp.float32)]
  else:
    out_type = [jax.ShapeDtypeStruct((2 * _N, _H), jnp.float32)]
  scratch = [
      pltpu.VMEM((_C,), jnp.int32),
      pltpu.VMEM((_C,), jnp.int32),
      pltpu.VMEM((_C, _H), jnp.float32),
      pltpu.VMEM_SHARED((_NP, _H), jnp.float32),
      pltpu.SemaphoreType.DMA,
  ]

  def body(src2_hbm, dst_hbm, z_s, zeros_hbm, agg_s,
           src_v, dst_v, rows_v, acc_s, sem):
    cid = lax.axis_index("c")
    tid = lax.axis_index("s")
    rbase = tid * _RPT

    pltpu.sync_copy(zeros_hbm, rows_v)

    def zero_blk(k, carry):
      pltpu.sync_copy(rows_v, acc_s.at[pl.ds(rbase + k * _C, _C)])
      return carry

    lax.fori_loop(0, _RPT // _C, zero_blk, 0)
    plsc.subcore_barrier()

    ebase = tid * _EPT
    sbase = cid * _E + ebase
    if with_deg:
      pltpu.sync_copy(z_s, rows_v)

      def chunk(i, carry):
        pltpu.sync_copy(dst_hbm.at[pl.ds(ebase + i * _C, _C)], dst_v)
        pltpu.sync_copy(rows_v, acc_s.at[dst_v], add=True)
        return carry
    else:

      def chunk(i, carry):
        pltpu.sync_copy(src2_hbm.at[pl.ds(sbase + i * _C, _C)], src_v)
        pltpu.sync_copy(dst_hbm.at[pl.ds(ebase + i * _C, _C)], dst_v)
        pltpu.async_copy(z_s.at[src_v], rows_v, sem).wait()
        pltpu.sync_copy(rows_v, acc_s.at[dst_v], add=True)
        return carry

    lax.fori_loop(0, _NCHUNK, chunk, 0)
    plsc.subcore_barrier()

    nch = jnp.where(tid == _NT - 1, (_N - (_NT - 1) * _RPT) // _C,
                    _RPT // _C)

    def out_blk(k, carry):
      r = rbase + k * _C
      pltpu.sync_copy(acc_s.at[pl.ds(r, _C)], rows_v)
      if with_deg:
        @pl.when(cid == 0)
        def _():
          pltpu.sync_copy(rows_v, agg_s.at[pl.ds(r, _C)])
      else:
        pltpu.sync_copy(rows_v, agg_s.at[pl.ds(cid * _N + r, _C)])
      return carry

    lax.fori_loop(0, nch, out_blk, 0)

  return pl.kernel(body, out_type=out_type, mesh=mesh, scratch_types=scratch)


_segsum_deg = _make_segsum(True)
_segsum = _make_segsum(False)


def _matmul_z(x, Wn):
  def body(x_ref, w_ref, z_ref):
    z_ref[...] = jnp.dot(x_ref[...], w_ref[...],
                         preferred_element_type=jnp.float32)

  nb = _N // _BM
  return pl.pallas_call(
      body,
      grid=(nb, 2),
      in_specs=[pl.BlockSpec((_BM, _D), lambda i, j: (i, 0)),
                pl.BlockSpec((_D, _H), lambda i, j: (0, j))],
      out_specs=pl.BlockSpec((_BM, _H), lambda i, j: (j * nb + i, 0)),
      out_shape=jax.ShapeDtypeStruct((2 * _N, _H), jnp.float32),
  )(x, Wn)


def _combine(x, agg_s, deg16, Ws, b2d, *, relu):
  nb = _N // _BM

  def body(x_ref, aa_ref, ab_ref, deg_ref, ws_ref, b_ref, h_ref):
    h = jnp.dot(x_ref[...], ws_ref[...], preferred_element_type=jnp.float32)
    inv = 1.0 / jnp.maximum(deg_ref[...][:, :1], 1.0)
    agg = jnp.concatenate([aa_ref[...], ab_ref[...]], axis=1)
    h = h + b_ref[...] + inv * agg
    if relu:
      h = jnp.maximum(h, 0.0)
    h_ref[...] = h

  return pl.pallas_call(
      body,
      grid=(nb,),
      in_specs=[pl.BlockSpec((_BM, _D), lambda i: (i, 0)),
                pl.BlockSpec((_BM, _H), lambda i: (i, 0)),
                pl.BlockSpec((_BM, _H), lambda i: (nb + i, 0)),
                pl.BlockSpec((_BM, _H), lambda i: (i, 0)),
                pl.BlockSpec((_D, _D), lambda i: (0, 0)),
                pl.BlockSpec((1, _D), lambda i: (0, 0))],
      out_specs=pl.BlockSpec((_BM, _D), lambda i: (i, 0)),
      out_shape=jax.ShapeDtypeStruct((_N, _D), jnp.float32),
  )(x, agg_s, agg_s, deg16, Ws, b2d)


def kernel(x, edge_index, Ws0, Wn0, b0, Ws1, Wn1, b1, Ws2, Wn2, b2):
  src = edge_index[0]
  dst = edge_index[1]
  src2 = jnp.concatenate([src, src + _N])
  zeros_big = jnp.zeros((_C, _H), jnp.float32)
  ones_big = jnp.ones((_C, _H), jnp.float32)

  z0 = _matmul_z(x, Wn0)
  deg, = _segsum_deg(src2, dst, ones_big, zeros_big)
  agg0, = _segsum(src2, dst, z0, zeros_big)
  h1 = _combine(x, agg0, deg, Ws0, b0.reshape(1, _D), relu=True)
  z1 = _matmul_z(h1, Wn1)
  agg1, = _segsum(src2, dst, z1, zeros_big)
  h2 = _combine(h1, agg1, deg, Ws1, b1.reshape(1, _D), relu=True)
  z2 = _matmul_z(h2, Wn2)
  agg2, = _segsum(src2, dst, z2, zeros_big)
  h3 = _combine(h2, agg2, deg, Ws2, b2.reshape(1, _D), relu=False)
  return h3


# speedup vs baseline: 4.6118x; 1.5738x over previous
import functools

import jax
import jax.numpy as jnp
from jax import lax
from jax.experimental import pallas as pl
from jax.experimental.pallas import tpu as pltpu
from jax.experimental.pallas import tpu_sc as plsc

_N = 10000
_NP = 10240
_E = 160000
_D = 256
_H = 128
_NT = 16
_EPT = _E // _NT
_C = 80
_NCHUNK = _EPT // _C
_RPT = _NP // _NT
_BM = 1000


def _make_segsum(with_deg):
  mesh = plsc.VectorSubcoreMesh(core_axis_name="c", subcore_axis_name="s")
  if with_deg:
    out_type = [jax.ShapeDtypeStruct((_N, _H), jnp.float32)]
  else:
    out_type = [jax.ShapeDtypeStruct((2 * _N, _H), jnp.float32)]
  scratch = [
      pltpu.VMEM((2, _C), jnp.int32),
      pltpu.VMEM((2, _C), jnp.int32),
      pltpu.VMEM((2, _C, _H), jnp.float32),
      pltpu.VMEM_SHARED((_NP, _H), jnp.float32),
      pltpu.SemaphoreType.DMA,
  ]

  def body(src2_hbm, dst_hbm, z_s, zeros_hbm, agg_s,
           src_v, dst_v, rows_v, acc_s, sem):
    cid = lax.axis_index("c")
    tid = lax.axis_index("s")
    rbase = tid * _RPT

    pltpu.sync_copy(zeros_hbm, rows_v.at[0])

    def zero_blk(k, carry):
      pltpu.sync_copy(rows_v.at[0], acc_s.at[pl.ds(rbase + k * _C, _C)])
      return carry

    lax.fori_loop(0, _RPT // _C, zero_blk, 0)
    plsc.subcore_barrier()

    ebase = tid * _EPT
    sbase = cid * _E + ebase
    if with_deg:
      pltpu.sync_copy(z_s, rows_v.at[0])
      pltpu.async_copy(dst_hbm.at[pl.ds(ebase, _C)], dst_v.at[0], sem)

      def pair(g, carry):
        for b in (0, 1):
          i = 2 * g + b
          nb = 1 - b
          pltpu.make_async_copy(dst_hbm.at[pl.ds(ebase + i * _C, _C)],
                                dst_v.at[b], sem).wait()
          pltpu.async_copy(dst_hbm.at[pl.ds(ebase + (i + 1) * _C, _C)],
                           dst_v.at[nb], sem)
          pltpu.sync_copy(rows_v.at[0], acc_s.at[dst_v.at[b]], add=True)
        return carry

      lax.fori_loop(0, (_NCHUNK - 1) // 2, pair, 0)
      last = _NCHUNK - 1
      pltpu.make_async_copy(dst_hbm.at[pl.ds(ebase + last * _C, _C)],
                            dst_v.at[last % 2], sem).wait()
      pltpu.sync_copy(rows_v.at[0], acc_s.at[dst_v.at[last % 2]], add=True)
    else:
      pltpu.sync_copy(src2_hbm.at[pl.ds(sbase, _C)], src_v.at[0])
      pltpu.sync_copy(dst_hbm.at[pl.ds(ebase, _C)], dst_v.at[0])
      pltpu.async_copy(z_s.at[src_v.at[0]], rows_v.at[0], sem)

      def pair(g, carry):
        for b in (0, 1):
          i = 2 * g + b
          nb = 1 - b
          pltpu.sync_copy(src2_hbm.at[pl.ds(sbase + (i + 1) * _C, _C)],
                          src_v.at[nb])
          pltpu.sync_copy(dst_hbm.at[pl.ds(ebase + (i + 1) * _C, _C)],
                          dst_v.at[nb])
          pltpu.make_async_copy(z_s.at[src_v.at[b]], rows_v.at[b],
                                sem).wait()
          pltpu.async_copy(z_s.at[src_v.at[nb]], rows_v.at[nb], sem)
          pltpu.sync_copy(rows_v.at[b], acc_s.at[dst_v.at[b]], add=True)
        return carry

      lax.fori_loop(0, (_NCHUNK - 1) // 2, pair, 0)
      last = _NCHUNK - 1
      pltpu.make_async_copy(z_s.at[src_v.at[last % 2]],
                            rows_v.at[last % 2], sem).wait()
      pltpu.sync_copy(rows_v.at[last % 2], acc_s.at[dst_v.at[last % 2]],
                      add=True)
    plsc.subcore_barrier()

    nch = jnp.where(tid == _NT - 1, (_N - (_NT - 1) * _RPT) // _C,
                    _RPT // _C)

    def out_blk(k, carry):
      r = rbase + k * _C
      pltpu.sync_copy(acc_s.at[pl.ds(r, _C)], rows_v.at[0])
      if with_deg:
        @pl.when(cid == 0)
        def _():
          pltpu.sync_copy(rows_v.at[0], agg_s.at[pl.ds(r, _C)])
      else:
        pltpu.sync_copy(rows_v.at[0], agg_s.at[pl.ds(cid * _N + r, _C)])
      return carry

    lax.fori_loop(0, nch, out_blk, 0)

  return pl.kernel(body, out_type=out_type, mesh=mesh, scratch_types=scratch)


_segsum_deg = _make_segsum(True)
_segsum = _make_segsum(False)


def _matmul_z(x, Wn):
  def body(x_ref, w_ref, z_ref):
    z_ref[...] = jnp.dot(x_ref[...], w_ref[...],
                         preferred_element_type=jnp.float32)

  nb = _N // _BM
  return pl.pallas_call(
      body,
      grid=(nb, 2),
      in_specs=[pl.BlockSpec((_BM, _D), lambda i, j: (i, 0)),
                pl.BlockSpec((_D, _H), lambda i, j: (0, j))],
      out_specs=pl.BlockSpec((_BM, _H), lambda i, j: (j * nb + i, 0)),
      out_shape=jax.ShapeDtypeStruct((2 * _N, _H), jnp.float32),
  )(x, Wn)


def _combine(x, agg_s, deg16, Ws, b2d, *, relu):
  nb = _N // _BM

  def body(x_ref, aa_ref, ab_ref, deg_ref, ws_ref, b_ref, h_ref):
    h = jnp.dot(x_ref[...], ws_ref[...], preferred_element_type=jnp.float32)
    inv = 1.0 / jnp.maximum(deg_ref[...][:, :1], 1.0)
    agg = jnp.concatenate([aa_ref[...], ab_ref[...]], axis=1)
    h = h + b_ref[...] + inv * agg
    if relu:
      h = jnp.maximum(h, 0.0)
    h_ref[...] = h

  return pl.pallas_call(
      body,
      grid=(nb,),
      in_specs=[pl.BlockSpec((_BM, _D), lambda i: (i, 0)),
                pl.BlockSpec((_BM, _H), lambda i: (i, 0)),
                pl.BlockSpec((_BM, _H), lambda i: (nb + i, 0)),
                pl.BlockSpec((_BM, _H), lambda i: (i, 0)),
                pl.BlockSpec((_D, _D), lambda i: (0, 0)),
                pl.BlockSpec((1, _D), lambda i: (0, 0))],
      out_specs=pl.BlockSpec((_BM, _D), lambda i: (i, 0)),
      out_shape=jax.ShapeDtypeStruct((_N, _D), jnp.float32),
  )(x, agg_s, agg_s, deg16, Ws, b2d)


def kernel(x, edge_index, Ws0, Wn0, b0, Ws1, Wn1, b1, Ws2, Wn2, b2):
  src = edge_index[0]
  dst = edge_index[1]
  src2 = jnp.concatenate([src, src + _N])
  zeros_big = jnp.zeros((_C, _H), jnp.float32)
  ones_big = jnp.ones((_C, _H), jnp.float32)

  z0 = _matmul_z(x, Wn0)
  deg, = _segsum_deg(src2, dst, ones_big, zeros_big)
  agg0, = _segsum(src2, dst, z0, zeros_big)
  h1 = _combine(x, agg0, deg, Ws0, b0.reshape(1, _D), relu=True)
  z1 = _matmul_z(h1, Wn1)
  agg1, = _segsum(src2, dst, z1, zeros_big)
  h2 = _combine(h1, agg1, deg, Ws1, b1.reshape(1, _D), relu=True)
  z2 = _matmul_z(h2, Wn2)
  agg2, = _segsum(src2, dst, z2, zeros_big)
  h3 = _combine(h2, agg2, deg, Ws2, b2.reshape(1, _D), relu=False)
  return h3


# speedup vs baseline: 4.6172x; 1.0012x over previous
import jax
import jax.numpy as jnp
from jax import lax
from jax.experimental import pallas as pl
from jax.experimental.pallas import tpu as pltpu
from jax.experimental.pallas import tpu_sc as plsc

_N = 10000
_NP = 10240
_E = 160000
_D = 256
_H = 128
_NT = 16
_EPT = _E // _NT
_C = 80
_NCHUNK = _EPT // _C
_RPT = _NP // _NT
_BM = 1000


def _make_segsum(with_deg):
  mesh = plsc.VectorSubcoreMesh(core_axis_name="c", subcore_axis_name="s")
  if with_deg:
    out_type = [jax.ShapeDtypeStruct((_N, _H), jnp.float32)]
  else:
    out_type = [jax.ShapeDtypeStruct((2 * _N, _H), jnp.float32)]
  scratch = [
      pltpu.VMEM((2, _C), jnp.int32),
      pltpu.VMEM((2, _C), jnp.int32),
      pltpu.VMEM((2, _C, _H), jnp.float32),
      pltpu.VMEM_SHARED((_NP, _H), jnp.float32),
      pltpu.SemaphoreType.DMA,
  ]

  def body(src2_hbm, dst_hbm, z_s, zeros_hbm, agg_s,
           src_v, dst_v, rows_v, acc_s, sem):
    cid = lax.axis_index("c")
    tid = lax.axis_index("s")
    rbase = tid * _RPT

    pltpu.sync_copy(zeros_hbm, rows_v.at[0])

    def zero_blk(k, carry):
      pltpu.sync_copy(rows_v.at[0], acc_s.at[pl.ds(rbase + k * _C, _C)])
      return carry

    lax.fori_loop(0, _RPT // _C, zero_blk, 0)
    plsc.subcore_barrier()

    ebase = tid * _EPT
    sbase = cid * _E + ebase
    if with_deg:
      pltpu.sync_copy(z_s, rows_v.at[0])
      pltpu.async_copy(dst_hbm.at[pl.ds(ebase, _C)], dst_v.at[0], sem)

      def pair(g, carry):
        for b in (0, 1):
          i = 2 * g + b
          nb = 1 - b
          pltpu.make_async_copy(dst_hbm.at[pl.ds(ebase + i * _C, _C)],
                                dst_v.at[b], sem).wait()
          pltpu.async_copy(dst_hbm.at[pl.ds(ebase + (i + 1) * _C, _C)],
                           dst_v.at[nb], sem)
          pltpu.sync_copy(rows_v.at[0], acc_s.at[dst_v.at[b]], add=True)
        return carry

      lax.fori_loop(0, (_NCHUNK - 1) // 2, pair, 0)
      last = _NCHUNK - 1
      pltpu.make_async_copy(dst_hbm.at[pl.ds(ebase + last * _C, _C)],
                            dst_v.at[last % 2], sem).wait()
      pltpu.sync_copy(rows_v.at[0], acc_s.at[dst_v.at[last % 2]], add=True)
    else:
      pltpu.sync_copy(src2_hbm.at[pl.ds(sbase, _C)], src_v.at[0])
      pltpu.sync_copy(dst_hbm.at[pl.ds(ebase, _C)], dst_v.at[0])
      pltpu.async_copy(z_s.at[src_v.at[0]], rows_v.at[0], sem)

      def pair(g, carry):
        for b in (0, 1):
          i = 2 * g + b
          nb = 1 - b
          pltpu.sync_copy(src2_hbm.at[pl.ds(sbase + (i + 1) * _C, _C)],
                          src_v.at[nb])
          pltpu.sync_copy(dst_hbm.at[pl.ds(ebase + (i + 1) * _C, _C)],
                          dst_v.at[nb])
          pltpu.make_async_copy(z_s.at[src_v.at[b]], rows_v.at[b],
                                sem).wait()
          pltpu.async_copy(z_s.at[src_v.at[nb]], rows_v.at[nb], sem)
          pltpu.sync_copy(rows_v.at[b], acc_s.at[dst_v.at[b]], add=True)
        return carry

      lax.fori_loop(0, (_NCHUNK - 1) // 2, pair, 0)
      last = _NCHUNK - 1
      pltpu.make_async_copy(z_s.at[src_v.at[last % 2]],
                            rows_v.at[last % 2], sem).wait()
      pltpu.sync_copy(rows_v.at[last % 2], acc_s.at[dst_v.at[last % 2]],
                      add=True)
    plsc.subcore_barrier()

    nch = jnp.where(tid == _NT - 1, (_N - (_NT - 1) * _RPT) // _C,
                    _RPT // _C)

    def out_blk(k, carry):
      r = rbase + k * _C
      pltpu.sync_copy(acc_s.at[pl.ds(r, _C)], rows_v.at[0])
      if with_deg:
        @pl.when(cid == 0)
        def _():
          pltpu.sync_copy(rows_v.at[0], agg_s.at[pl.ds(r, _C)])
      else:
        pltpu.sync_copy(rows_v.at[0], agg_s.at[pl.ds(cid * _N + r, _C)])
      return carry

    lax.fori_loop(0, nch, out_blk, 0)

  return pl.kernel(body, out_type=out_type, mesh=mesh, scratch_types=scratch)


_segsum_deg = _make_segsum(True)
_segsum = _make_segsum(False)


def _matmul_z(x, Wn):
  def body(x_ref, w_ref, z_ref):
    z_ref[...] = jnp.dot(x_ref[...], w_ref[...],
                         preferred_element_type=jnp.float32)

  nb = _N // _BM
  return pl.pallas_call(
      body,
      grid=(nb, 2),
      in_specs=[pl.BlockSpec((_BM, _D), lambda i, j: (i, 0)),
                pl.BlockSpec((_D, _H), lambda i, j: (0, j))],
      out_specs=pl.BlockSpec((_BM, _H), lambda i, j: (j * nb + i, 0)),
      out_shape=jax.ShapeDtypeStruct((2 * _N, _H), jnp.float32),
  )(x, Wn)


def _combine(x, agg_s, deg, Ws, b2d, *, relu):
  nb = _N // _BM

  def body(x_ref, aa_ref, ab_ref, deg_ref, ws_ref, b_ref, h_ref):
    h = jnp.dot(x_ref[...], ws_ref[...], preferred_element_type=jnp.float32)
    inv = 1.0 / jnp.maximum(deg_ref[...][:, :1], 1.0)
    agg = jnp.concatenate([aa_ref[...], ab_ref[...]], axis=1)
    h = h + b_ref[...] + inv * agg
    if relu:
      h = jnp.maximum(h, 0.0)
    h_ref[...] = h

  return pl.pallas_call(
      body,
      grid=(nb,),
      in_specs=[pl.BlockSpec((_BM, _D), lambda i: (i, 0)),
                pl.BlockSpec((_BM, _H), lambda i: (i, 0)),
                pl.BlockSpec((_BM, _H), lambda i: (nb + i, 0)),
                pl.BlockSpec((_BM, _H), lambda i: (i, 0)),
                pl.BlockSpec((_D, _D), lambda i: (0, 0)),
                pl.BlockSpec((1, _D), lambda i: (0, 0))],
      out_specs=pl.BlockSpec((_BM, _D), lambda i: (i, 0)),
      out_shape=jax.ShapeDtypeStruct((_N, _D), jnp.float32),
  )(x, agg_s, agg_s, deg, Ws, b2d)


def kernel(x, edge_index, Ws0, Wn0, b0, Ws1, Wn1, b1, Ws2, Wn2, b2):
  src = edge_index[0]
  dst = edge_index[1]
  src2 = jnp.concatenate([src, src + _N])
  zeros_big = jnp.zeros((_C, _H), jnp.float32)
  ones_big = jnp.ones((_C, _H), jnp.float32)

  z0 = _matmul_z(x, Wn0)
  deg, = _segsum_deg(src2, dst, ones_big, zeros_big)
  agg0, = _segsum(src2, dst, z0, zeros_big)
  h1 = _combine(x, agg0, deg, Ws0, b0.reshape(1, _D), relu=True)
  z1 = _matmul_z(h1, Wn1)
  agg1, = _segsum(src2, dst, z1, zeros_big)
  h2 = _combine(h1, agg1, deg, Ws1, b1.reshape(1, _D), relu=True)
  z2 = _matmul_z(h2, Wn2)
  agg2, = _segsum(src2, dst, z2, zeros_big)
  h3 = _combine(h2, agg2, deg, Ws2, b2.reshape(1, _D), relu=False)
  return h3
